# baseline (speedup 1.0000x reference)
import functools

import numpy as np
import jax
import jax.numpy as jnp
from jax import lax
from jax.experimental import pallas as pl
from jax.experimental.pallas import tpu as pltpu
from jax.experimental.pallas import tpu_sc as plsc

RES = 0.5
NPTS = 48
ORIGIN = -11.75
E2 = 0.1353352832366127
W = 13
NXY = W * W
NDROW = 2 * NXY
T = 14
NROW_PAD = 384
NCHUNK = NROW_PAD // 128
ZROWS = 3
ACC_ROWS = T * NPTS * NPTS * ZROWS
STRIPE = ACC_ROWS // 16
ZCH = 12
ZCHR = STRIPE // ZCH


def _pattern() -> np.ndarray:
    p = np.zeros((NROW_PAD,), np.int32)
    i = np.arange(NDROW)
    xy = i // 2
    zb = i % 2
    x = xy // W
    y = xy % W
    p[:NDROW] = (x * NPTS + y) * ZROWS + zb
    return p.reshape(NCHUNK, 128)


_P_HOST = _pattern()


def _splat(vec, j):
    return vec.at[jnp.full((16,), j, jnp.int32)].get(
        mode="promise_in_bounds")


def _sc_body(atoms, pfull, out, rec, dens, src, idx, pvm, zbuf, acc):
    c = lax.axis_index("c")
    s = lax.axis_index("s")
    lanes = lax.iota(jnp.int32, 16)
    zero16 = jnp.zeros((16,), jnp.float32)

    pltpu.sync_copy(pfull, pvm)

    def zb_body(i, _):
        zbuf[i, :] = zero16
        return ()
    lax.fori_loop(0, ZCHR, zb_body, ())
    for j in range(NROW_PAD - NDROW):
        src[NDROW + j, :] = zero16

    def atom_body(k, carry):
        b = carry
        a = b * 128 + s * 8 + k
        pltpu.sync_copy(atoms.at[a], rec)
        r0 = rec[0:16]
        r1 = rec[16:32]
        cx = _splat(r0, 0)
        cy = _splat(r0, 1)
        cz = _splat(r0, 2)
        r = _splat(r0, 3)
        r2 = r * r
        inv_r2 = 1.0 / r2
        neg2 = -2.0 * inv_r2
        c1 = (4.0 * E2) * inv_r2
        c2 = (12.0 * E2) / r
        q225 = 2.25 * r2

        def start(cv):
            tx = jnp.clip((cv - 3.0 - ORIGIN) * 2.0, -1.0, 40.0)
            ti = tx.astype(jnp.int32)
            ti = ti + jnp.where(ti.astype(jnp.float32) < tx, 1, 0)
            return jnp.clip(ti, 0, NPTS - W)

        ix0 = start(cx)
        iy0 = start(cy)
        iz0 = start(cz)
        zb0 = jnp.where(iz0 >= 16, 1, 0)
        zp = (zb0 * 16).astype(jnp.float32)

        axv = ORIGIN + RES * (ix0 + lanes).astype(jnp.float32)
        dxv = axv - cx
        dx2v = dxv * dxv
        ayv = ORIGIN + RES * (iy0 + lanes).astype(jnp.float32)
        dyv = ayv - cy
        dy2v = dyv * dyv
        lf = lanes.astype(jnp.float32)
        za = ORIGIN + RES * (zp + lf)
        zb_ = ORIGIN + RES * (zp + 16.0 + lf)
        dz2a = (za - cz) * (za - cz)
        dz2b = (zb_ - cz) * (zb_ - cz)

        def density(d2):
            ga = jnp.exp(neg2 * d2)
            yi = lax.bitcast_convert_type(d2, jnp.int32)
            yi = 0x5F3759DF - lax.shift_right_arithmetic(yi, 1)
            y = lax.bitcast_convert_type(yi, jnp.float32)
            hw = (d2 * y) * y
            y = y * (1.5 - 0.5 * hw)
            hw = (d2 * y) * y
            y = y * (1.5 - 0.5 * hw)
            d = d2 * y
            q = c1 * d2 - c2 * d + 9.0 * E2
            return jnp.where(d2 < r2, ga,
                             jnp.where(d2 < q225, q, 0.0))

        def x_body(x, _):
            d2x = _splat(dx2v, x)

            def y_body(y, _):
                d2xy = d2x + _splat(dy2v, y)
                i2 = (x * W + y) * 2
                dens[i2, :] = density(d2xy + dz2a)
                dens[i2 + 1, :] = density(d2xy + dz2b)
                return ()
            lax.fori_loop(0, W, y_body, ())
            return ()
        lax.fori_loop(0, W, x_body, ())

        off = ix0 * (NPTS * ZROWS) + iy0 * ZROWS + zb0
        wv = rec[4:20]

        def t_body(t, _):
            w = _splat(wv, t)

            def w_body(i, _):
                src[i, :] = w * dens[i, :]
                return ()
            lax.fori_loop(0, NDROW, w_body, ())

            offt = off + t * (NPTS * NPTS * ZROWS)

            def i_body(v, _):
                g = v // 8
                l = v % 8
                idx[g, pl.ds(l * 16, 16)] = pvm[g, pl.ds(l * 16, 16)] + offt
                return ()
            lax.fori_loop(0, NCHUNK * 8, i_body, ())

            for g in range(NCHUNK):
                pltpu.sync_copy(src.at[pl.ds(g * 128, 128)],
                                acc.at[idx.at[g]], add=True)
            return ()
        lax.fori_loop(0, T, t_body, ())
        return carry

    for bi in range(4):
        b = c * 4 + bi
        base = s * STRIPE
        for j in range(ZCH):
            pltpu.sync_copy(zbuf, acc.at[pl.ds(base + j * ZCHR, ZCHR)])
        plsc.subcore_barrier()
        lax.fori_loop(0, 8, atom_body, b)
        plsc.subcore_barrier()
        for j in range(ZCH):
            sl = pl.ds(base + j * ZCHR, ZCHR)
            pltpu.sync_copy(acc.at[sl], out.at[b].at[sl])
        plsc.subcore_barrier()


@jax.jit
def kernel(coords, types, radii):
    B, N, _ = coords.shape
    flat = B * N
    atoms = jnp.concatenate(
        [coords.reshape(flat, 3), radii.reshape(flat, 1),
         types.reshape(flat, T),
         jnp.zeros((flat, 32 - 4 - T), jnp.float32)], axis=1)
    pfull = jnp.asarray(_P_HOST)

    mesh = plsc.VectorSubcoreMesh(core_axis_name="c", subcore_axis_name="s")
    run = pl.kernel(
        _sc_body,
        mesh=mesh,
        compiler_params=pltpu.CompilerParams(use_tc_tiling_on_sc=False),
        out_type=jax.ShapeDtypeStruct((B, ACC_ROWS, 16), jnp.float32),
        scratch_types=[
            pltpu.VMEM((32,), jnp.float32),
            pltpu.VMEM((NDROW, 16), jnp.float32),
            pltpu.VMEM((NROW_PAD, 16), jnp.float32),
            pltpu.VMEM((NCHUNK, 128), jnp.int32),
            pltpu.VMEM((NCHUNK, 128), jnp.int32),
            pltpu.VMEM((ZCHR, 16), jnp.float32),
            pltpu.VMEM_SHARED((ACC_ROWS, 16), jnp.float32),
        ],
    )
    out = run(atoms, pfull)
    return out.reshape(B, T, NPTS, NPTS, NPTS)
